# Initial kernel scaffold; baseline (speedup 1.0000x reference)
import jax
import jax.numpy as jnp
from jax.experimental import pallas as pl


def kernel(indices, W_full, p):
    raise NotImplementedError("write your pallas kernel here")


# speedup vs baseline: 2.6653x; 2.6653x over previous
import functools

import jax
import jax.numpy as jnp
from jax import lax
from jax.experimental import pallas as pl
from jax.experimental.pallas import tpu as pltpu
from jax.experimental.pallas import tpu_sc as plsc

_WORD_COUNT = 1000000
_NUM_BUCKETS = 100000
_EMB = 16
_PRIMES = (65521, 60013)
_LANES = 16
_NW = 32
_GBLK = 128
_CHUNK = 1024
_NG = _CHUNK // _GBLK


def _exact_mod(wf, prime):
    q = (wf * (1.0 / prime)).astype(jnp.int32).astype(jnp.float32)
    r = wf - q * float(prime)
    r = jnp.where(r < 0.0, r + float(prime), r)
    r = jnp.where(r >= float(prime), r - float(prime), r)
    return r.astype(jnp.int32)


def kernel(indices, W_full, p):
    B, L = indices.shape
    n = B * L
    per_w = n // _NW
    nblk = per_w // _CHUNK
    idx_flat = indices.reshape(n)
    p_flat = p.reshape(-1)

    mesh = plsc.VectorSubcoreMesh(core_axis_name="c", subcore_axis_name="s")

    @functools.partial(
        pl.kernel,
        mesh=mesh,
        compiler_params=pltpu.CompilerParams(use_tc_tiling_on_sc=False),
        out_type=jax.ShapeDtypeStruct((n, _EMB), jnp.float32),
        scratch_types=[
            pltpu.VMEM((_CHUNK,), jnp.int32),
            pltpu.VMEM((_NG, _GBLK), jnp.int32),
            pltpu.VMEM((_NG, _GBLK), jnp.int32),
            pltpu.VMEM((_NG, _GBLK), jnp.int32),
            pltpu.VMEM((_NG, _GBLK), jnp.int32),
            pltpu.VMEM((_CHUNK, _EMB), jnp.float32),
            pltpu.VMEM((_CHUNK, _EMB), jnp.float32),
            pltpu.VMEM((_CHUNK,), jnp.float32),
            pltpu.VMEM((_CHUNK,), jnp.float32),
            pltpu.SemaphoreType.DMA,
        ],
    )
    def run(idx_hbm, w_hbm, p_hbm, out_hbm,
            widx, i0, i1, ip0, ip1, r0, r1, p0b, p1b, sem):
        wid = lax.axis_index("s") * 2 + lax.axis_index("c")
        base = wid * per_w

        def block(blk, carry):
            off = base + blk * _CHUNK
            pltpu.sync_copy(idx_hbm.at[pl.ds(off, _CHUNK)], widx)

            def idx_row(r, carry2):
                for g in range(_GBLK // _LANES):
                    w = widx[pl.ds(r * _GBLK + g * _LANES, _LANES)]
                    wf = w.astype(jnp.float32)
                    nz = w != 0
                    a0 = jnp.where(nz, _exact_mod(wf, _PRIMES[0]), 0)
                    a1 = jnp.where(nz, _exact_mod(wf, _PRIMES[1]), 0)
                    wp = w + 3
                    ap = jnp.where(wp >= _WORD_COUNT, wp - _WORD_COUNT, wp)
                    ap2 = ap + ap
                    sl = pl.ds(g * _LANES, _LANES)
                    i0[r, sl] = a0
                    i1[r, sl] = a1
                    ip0[r, sl] = ap2
                    ip1[r, sl] = ap2 + 1
                return carry2

            lax.fori_loop(0, _NG, idx_row, 0)

            copies = []
            for j in range(_NG):
                dst = pl.ds(j * _GBLK, _GBLK)
                copies.append(pltpu.async_copy(w_hbm.at[i0.at[j]], r0.at[dst], sem))
                copies.append(pltpu.async_copy(w_hbm.at[i1.at[j]], r1.at[dst], sem))
                copies.append(pltpu.async_copy(p_hbm.at[ip0.at[j]], p0b.at[dst], sem))
                copies.append(pltpu.async_copy(p_hbm.at[ip1.at[j]], p1b.at[dst], sem))
            for c in copies:
                c.wait()

            def mul_body(t16, carry3):
                t0 = t16 * _LANES
                p0v = p0b[pl.ds(t0, _LANES)]
                p1v = p1b[pl.ds(t0, _LANES)]
                for u in range(_LANES):
                    t = t0 + u
                    r0[t] = r0[t] * p0v[u] + r1[t] * p1v[u]
                return carry3

            lax.fori_loop(0, _CHUNK // _LANES, mul_body, 0)

            pltpu.sync_copy(r0, out_hbm.at[pl.ds(off, _CHUNK)])
            return carry

        lax.fori_loop(0, nblk, block, 0)

    out = run(idx_flat, W_full, p_flat)
    return out.reshape(B, L, _EMB)


# speedup vs baseline: 3.1573x; 1.1846x over previous
import functools

import jax
import jax.numpy as jnp
from jax import lax
from jax.experimental import pallas as pl
from jax.experimental.pallas import tpu as pltpu
from jax.experimental.pallas import tpu_sc as plsc

_WORD_COUNT = 1000000
_NUM_BUCKETS = 100000
_EMB = 16
_PRIMES = (65521, 60013)
_LANES = 16
_NW = 32
_GBLK = 128
_CHUNK = 1024
_NG = _CHUNK // _GBLK


def _exact_mod(wf, prime):
    q = (wf * (1.0 / prime)).astype(jnp.int32).astype(jnp.float32)
    r = wf - q * float(prime)
    r = jnp.where(r < 0.0, r + float(prime), r)
    r = jnp.where(r >= float(prime), r - float(prime), r)
    return r.astype(jnp.int32)


def kernel(indices, W_full, p):
    B, L = indices.shape
    n = B * L
    per_w = n // _NW
    nblk = per_w // _CHUNK
    idx_flat = indices.reshape(n)
    p_flat = p.reshape(-1)

    mesh = plsc.VectorSubcoreMesh(core_axis_name="c", subcore_axis_name="s")

    @functools.partial(
        pl.kernel,
        mesh=mesh,
        compiler_params=pltpu.CompilerParams(use_tc_tiling_on_sc=False),
        out_type=jax.ShapeDtypeStruct((n * _EMB,), jnp.float32),
        scratch_types=[
            pltpu.VMEM((_CHUNK,), jnp.int32),
            pltpu.VMEM((_NG, _GBLK), jnp.int32),
            pltpu.VMEM((_NG, _GBLK), jnp.int32),
            pltpu.VMEM((_NG, _GBLK), jnp.int32),
            pltpu.VMEM((_NG, _GBLK), jnp.int32),
            pltpu.VMEM((_CHUNK, _EMB), jnp.float32),
            pltpu.VMEM((_CHUNK, _EMB), jnp.float32),
            pltpu.VMEM((_CHUNK,), jnp.float32),
            pltpu.VMEM((_CHUNK,), jnp.float32),
            pltpu.VMEM((_CHUNK * _EMB,), jnp.float32),
            pltpu.SemaphoreType.DMA,
        ],
    )
    def run(idx_hbm, w_hbm, p_hbm, out_hbm,
            widx, i0, i1, ip0, ip1, r0, r1, p0b, p1b, ob, sem):
        wid = lax.axis_index("s") * 2 + lax.axis_index("c")
        base = wid * per_w

        def block(blk, carry):
            off = base + blk * _CHUNK
            pltpu.sync_copy(idx_hbm.at[pl.ds(off, _CHUNK)], widx)

            def idx_row(r, carry2):
                for g in range(_GBLK // _LANES):
                    w = widx[pl.ds(r * _GBLK + g * _LANES, _LANES)]
                    wf = w.astype(jnp.float32)
                    nz = w != 0
                    a0 = jnp.where(nz, _exact_mod(wf, _PRIMES[0]), 0)
                    a1 = jnp.where(nz, _exact_mod(wf, _PRIMES[1]), 0)
                    wp = w + 3
                    ap = jnp.where(wp >= _WORD_COUNT, wp - _WORD_COUNT, wp)
                    ap2 = ap + ap
                    sl = pl.ds(g * _LANES, _LANES)
                    i0[r, sl] = a0
                    i1[r, sl] = a1
                    ip0[r, sl] = ap2
                    ip1[r, sl] = ap2 + 1
                return carry2

            lax.fori_loop(0, _NG, idx_row, 0)

            copies = []
            for j in range(_NG):
                dst = pl.ds(j * _GBLK, _GBLK)
                copies.append(pltpu.async_copy(w_hbm.at[i0.at[j]], r0.at[dst], sem))
                copies.append(pltpu.async_copy(w_hbm.at[i1.at[j]], r1.at[dst], sem))
                copies.append(pltpu.async_copy(p_hbm.at[ip0.at[j]], p0b.at[dst], sem))
                copies.append(pltpu.async_copy(p_hbm.at[ip1.at[j]], p1b.at[dst], sem))
            for c in copies:
                c.wait()

            def mul_body(t16, carry3):
                t0 = t16 * _LANES
                p0v = p0b[pl.ds(t0, _LANES)]
                p1v = p1b[pl.ds(t0, _LANES)]
                for u in range(_LANES):
                    t = t0 + u
                    ob[pl.ds(t * _EMB, _EMB)] = r0[t] * p0v[u] + r1[t] * p1v[u]
                return carry3

            lax.fori_loop(0, _CHUNK // _LANES, mul_body, 0)

            pltpu.sync_copy(ob, out_hbm.at[pl.ds(off * _EMB, _CHUNK * _EMB)])
            return carry

        lax.fori_loop(0, nblk, block, 0)

    out = run(idx_flat, W_full, p_flat)
    return out.reshape(B, L, _EMB)


# speedup vs baseline: 8.5270x; 2.7007x over previous
import functools

import jax
import jax.numpy as jnp
from jax import lax
from jax.experimental import pallas as pl
from jax.experimental.pallas import tpu as pltpu
from jax.experimental.pallas import tpu_sc as plsc

_WORD_COUNT = 1000000
_NUM_BUCKETS = 100000
_EMB = 16
_PRIMES = (65521, 60013)
_LANES = 16
_NW = 32
_GBLK = 128
_CHUNK = 1024
_NG = _CHUNK // _GBLK


def _exact_mod(wf, prime):
    q = (wf * (1.0 / prime)).astype(jnp.int32).astype(jnp.float32)
    r = wf - q * float(prime)
    r = jnp.where(r < 0.0, r + float(prime), r)
    r = jnp.where(r >= float(prime), r - float(prime), r)
    return r.astype(jnp.int32)


def kernel(indices, W_full, p):
    B, L = indices.shape
    n = B * L
    per_w = n // _NW
    nblk = per_w // _CHUNK
    idx_flat = indices.reshape(n)
    p_flat = p.T.reshape(-1)

    mesh = plsc.VectorSubcoreMesh(core_axis_name="c", subcore_axis_name="s")

    @functools.partial(
        pl.kernel,
        mesh=mesh,
        compiler_params=pltpu.CompilerParams(use_tc_tiling_on_sc=False),
        out_type=jax.ShapeDtypeStruct((n * _EMB,), jnp.float32),
        scratch_types=[
            pltpu.VMEM((_CHUNK,), jnp.int32),
            pltpu.VMEM((_NG, _GBLK), jnp.int32),
            pltpu.VMEM((_NG, _GBLK), jnp.int32),
            pltpu.VMEM((_NG, _GBLK), jnp.int32),
            pltpu.VMEM((_NG, _GBLK), jnp.int32),
            pltpu.VMEM((_CHUNK, _EMB), jnp.float32),
            pltpu.VMEM((_CHUNK, _EMB), jnp.float32),
            pltpu.VMEM((_CHUNK,), jnp.float32),
            pltpu.VMEM((_CHUNK,), jnp.float32),
            pltpu.VMEM((_CHUNK * _EMB,), jnp.float32),
            pltpu.SemaphoreType.DMA,
        ],
    )
    def run(idx_hbm, w_hbm, p_hbm, out_hbm,
            widx, i0, i1, ip0, ip1, r0, r1, p0b, p1b, ob, sem):
        wid = lax.axis_index("s") * 2 + lax.axis_index("c")
        base = wid * per_w

        def block(blk, carry):
            off = base + blk * _CHUNK
            pltpu.sync_copy(idx_hbm.at[pl.ds(off, _CHUNK)], widx)

            def idx_row(r, carry2):
                for g in range(_GBLK // _LANES):
                    w = widx[pl.ds(r * _GBLK + g * _LANES, _LANES)]
                    wf = w.astype(jnp.float32)
                    nz = w != 0
                    a0 = jnp.where(nz, _exact_mod(wf, _PRIMES[0]), 0)
                    a1 = jnp.where(nz, _exact_mod(wf, _PRIMES[1]), 0)
                    wp = w + 3
                    ap = jnp.where(wp >= _WORD_COUNT, wp - _WORD_COUNT, wp)
                    sl = pl.ds(g * _LANES, _LANES)
                    i0[r, sl] = a0
                    i1[r, sl] = a1
                    ip0[r, sl] = ap
                    ip1[r, sl] = ap + _WORD_COUNT
                return carry2

            lax.fori_loop(0, _NG, idx_row, 0)

            copies = []
            for j in range(_NG):
                dst = pl.ds(j * _GBLK, _GBLK)
                copies.append(pltpu.async_copy(w_hbm.at[i0.at[j]], r0.at[dst], sem))
                copies.append(pltpu.async_copy(w_hbm.at[i1.at[j]], r1.at[dst], sem))
                copies.append(pltpu.async_copy(p_hbm.at[ip0.at[j]], p0b.at[dst], sem))
                copies.append(pltpu.async_copy(p_hbm.at[ip1.at[j]], p1b.at[dst], sem))
            for c in copies:
                c.wait()

            def mul_body(t16, carry3):
                t0 = t16 * _LANES
                p0v = p0b[pl.ds(t0, _LANES)]
                p1v = p1b[pl.ds(t0, _LANES)]
                for u in range(_LANES):
                    t = t0 + u
                    ob[pl.ds(t * _EMB, _EMB)] = r0[t] * p0v[u] + r1[t] * p1v[u]
                return carry3

            lax.fori_loop(0, _CHUNK // _LANES, mul_body, 0)

            pltpu.sync_copy(ob, out_hbm.at[pl.ds(off * _EMB, _CHUNK * _EMB)])
            return carry

        lax.fori_loop(0, nblk, block, 0)

    out = run(idx_flat, W_full, p_flat)
    return out.reshape(B, L, _EMB)


# speedup vs baseline: 10.4233x; 1.2224x over previous
import functools

import jax
import jax.numpy as jnp
from jax import lax
from jax.experimental import pallas as pl
from jax.experimental.pallas import tpu as pltpu
from jax.experimental.pallas import tpu_sc as plsc

_WORD_COUNT = 1000000
_NUM_BUCKETS = 100000
_EMB = 16
_PRIMES = (65521, 60013)
_LANES = 16
_NW = 32
_GBLK = 128
_BT_PER_W = 4
_CHUNK = _BT_PER_W * _GBLK


def _exact_mod(wf, prime):
    q = (wf * (1.0 / prime)).astype(jnp.int32).astype(jnp.float32)
    r = wf - q * float(prime)
    r = jnp.where(r < 0.0, r + float(prime), r)
    r = jnp.where(r >= float(prime), r - float(prime), r)
    return r.astype(jnp.int32)


def kernel(indices, W_full, p):
    B, L = indices.shape
    n = B * L
    nbt = B // _GBLK
    ne2 = _EMB // 8
    idx_t = indices.T.reshape(n)
    p_flat = p.T.reshape(-1)

    mesh = plsc.VectorSubcoreMesh(core_axis_name="c", subcore_axis_name="s")

    @functools.partial(
        pl.kernel,
        mesh=mesh,
        compiler_params=pltpu.CompilerParams(
            use_tc_tiling_on_sc=False, needs_layout_passes=False),
        out_type=jax.ShapeDtypeStruct((L * ne2, nbt * 8 * _GBLK), jnp.float32),
        scratch_types=[
            pltpu.VMEM((_CHUNK,), jnp.int32),
            pltpu.VMEM((_BT_PER_W, _GBLK), jnp.int32),
            pltpu.VMEM((_BT_PER_W, _GBLK), jnp.int32),
            pltpu.VMEM((_BT_PER_W, _GBLK), jnp.int32),
            pltpu.VMEM((_BT_PER_W, _GBLK), jnp.int32),
            pltpu.VMEM((_CHUNK, _EMB), jnp.float32),
            pltpu.VMEM((_CHUNK, _EMB), jnp.float32),
            pltpu.VMEM((_CHUNK,), jnp.float32),
            pltpu.VMEM((_CHUNK,), jnp.float32),
            pltpu.VMEM((ne2 * _BT_PER_W * 8 * _GBLK,), jnp.float32),
            pltpu.SemaphoreType.DMA,
        ],
    )
    def run(idx_hbm, w_hbm, p_hbm, out_hbm,
            widx, i0, i1, ip0, ip1, r0, r1, p0b, p1b, obf, sem):
        wid = lax.axis_index("s") * 2 + lax.axis_index("c")
        base_b = wid * _CHUNK
        hblk = 8 * _GBLK * _BT_PER_W
        lane16 = lax.iota(jnp.int32, _LANES)
        cvec = (lane16 >> 3) * hblk + (lane16 & 7) * _GBLK

        def hist_body(l, carry):
            pltpu.sync_copy(idx_hbm.at[pl.ds(l * B + base_b, _CHUNK)], widx)

            def idx_row(r, carry2):
                for g in range(_GBLK // _LANES):
                    w = widx[pl.ds(r * _GBLK + g * _LANES, _LANES)]
                    wf = w.astype(jnp.float32)
                    nz = w != 0
                    a0 = jnp.where(nz, _exact_mod(wf, _PRIMES[0]), 0)
                    a1 = jnp.where(nz, _exact_mod(wf, _PRIMES[1]), 0)
                    wp = w + 3
                    ap = jnp.where(wp >= _WORD_COUNT, wp - _WORD_COUNT, wp)
                    sl = pl.ds(g * _LANES, _LANES)
                    i0[r, sl] = a0
                    i1[r, sl] = a1
                    ip0[r, sl] = ap
                    ip1[r, sl] = ap + _WORD_COUNT
                return carry2

            lax.fori_loop(0, _BT_PER_W, idx_row, 0)

            copies = []
            for j in range(_BT_PER_W):
                dst = pl.ds(j * _GBLK, _GBLK)
                copies.append(pltpu.async_copy(w_hbm.at[i0.at[j]], r0.at[dst], sem))
                copies.append(pltpu.async_copy(w_hbm.at[i1.at[j]], r1.at[dst], sem))
                copies.append(pltpu.async_copy(p_hbm.at[ip0.at[j]], p0b.at[dst], sem))
                copies.append(pltpu.async_copy(p_hbm.at[ip1.at[j]], p1b.at[dst], sem))
            for c in copies:
                c.wait()

            for k in range(_BT_PER_W):
                def mul_body(sub, carry3, k=k):
                    t0 = k * _GBLK + sub * _LANES
                    p0v = p0b[pl.ds(t0, _LANES)]
                    p1v = p1b[pl.ds(t0, _LANES)]
                    for u in range(_LANES):
                        t = t0 + u
                        vrow = r0[t] * p0v[u] + r1[t] * p1v[u]
                        pos = cvec + (k * (8 * _GBLK) + sub * _LANES + u)
                        plsc.store_scatter(obf, [pos], vrow)
                    return carry3

                lax.fori_loop(0, _GBLK // _LANES, mul_body, 0)

            for e2 in range(ne2):
                pltpu.sync_copy(
                    obf.at[pl.ds(e2 * hblk, hblk)],
                    out_hbm.at[l * ne2 + e2, pl.ds(wid * hblk, hblk)],
                )
            return carry

        lax.fori_loop(0, L, hist_body, 0)

    out = run(idx_t, W_full, p_flat)
    return (out.reshape(L, ne2, nbt, 8, _GBLK)
            .transpose(2, 4, 0, 1, 3)
            .reshape(B, L, _EMB))


# speedup vs baseline: 14.9828x; 1.4374x over previous
import functools

import jax
import jax.numpy as jnp
from jax import lax
from jax.experimental import pallas as pl
from jax.experimental.pallas import tpu as pltpu
from jax.experimental.pallas import tpu_sc as plsc

_WORD_COUNT = 1000000
_NUM_BUCKETS = 100000
_EMB = 16
_PRIMES = (65521, 60013)
_LANES = 16
_NW = 32
_GBLK = 128
_BT_PER_W = 4
_CHUNK = _BT_PER_W * _GBLK
_HBLK = 8 * _GBLK * _BT_PER_W


def _exact_mod(wf, prime):
    q = (wf * (1.0 / prime)).astype(jnp.int32).astype(jnp.float32)
    r = wf - q * float(prime)
    r = jnp.where(r < 0.0, r + float(prime), r)
    r = jnp.where(r >= float(prime), r - float(prime), r)
    return r.astype(jnp.int32)


def kernel(indices, W_full, p):
    B, L = indices.shape
    n = B * L
    nbt = B // _GBLK
    ne2 = _EMB // 8
    idx_t = indices.T.reshape(n)
    p_flat = p.T.reshape(-1)

    mesh = plsc.VectorSubcoreMesh(core_axis_name="c", subcore_axis_name="s")

    nbuf = 2
    scratch = (
        [pltpu.VMEM((_CHUNK,), jnp.int32) for _ in range(nbuf)]
        + [pltpu.VMEM((_BT_PER_W, _GBLK), jnp.int32) for _ in range(4 * nbuf)]
        + [pltpu.VMEM((_CHUNK, _EMB), jnp.float32) for _ in range(2 * nbuf)]
        + [pltpu.VMEM((_CHUNK,), jnp.float32) for _ in range(2 * nbuf)]
        + [pltpu.VMEM((ne2 * _HBLK,), jnp.float32) for _ in range(nbuf)]
        + [pltpu.SemaphoreType.DMA for _ in range(3 * nbuf)]
    )

    @functools.partial(
        pl.kernel,
        mesh=mesh,
        compiler_params=pltpu.CompilerParams(
            use_tc_tiling_on_sc=False, needs_layout_passes=False),
        out_type=jax.ShapeDtypeStruct((L * ne2, nbt * 8 * _GBLK), jnp.float32),
        scratch_types=scratch,
    )
    def run(idx_hbm, w_hbm, p_hbm, out_hbm, *s):
        widx = s[0:2]
        i0 = s[2:4]
        i1 = s[4:6]
        ip0 = s[6:8]
        ip1 = s[8:10]
        r0 = s[10:12]
        r1 = s[12:14]
        p0b = s[14:16]
        p1b = s[16:18]
        obf = s[18:20]
        isem = s[20:22]
        gsem = s[22:24]
        wsem = s[24:26]

        wid = lax.axis_index("s") * 2 + lax.axis_index("c")
        base_b = wid * _CHUNK
        lane16 = lax.iota(jnp.int32, _LANES)
        cvec = (lane16 >> 3) * _HBLK + (lane16 & 7) * _GBLK

        def idx_compute(b):
            def idx_row(r, carry2):
                for g in range(_GBLK // _LANES):
                    w = widx[b][pl.ds(r * _GBLK + g * _LANES, _LANES)]
                    wf = w.astype(jnp.float32)
                    nz = w != 0
                    a0 = jnp.where(nz, _exact_mod(wf, _PRIMES[0]), 0)
                    a1 = jnp.where(nz, _exact_mod(wf, _PRIMES[1]), 0)
                    wp = w + 3
                    ap = jnp.where(wp >= _WORD_COUNT, wp - _WORD_COUNT, wp)
                    sl = pl.ds(g * _LANES, _LANES)
                    i0[b][r, sl] = a0
                    i1[b][r, sl] = a1
                    ip0[b][r, sl] = ap
                    ip1[b][r, sl] = ap + _WORD_COUNT
                return carry2

            lax.fori_loop(0, _BT_PER_W, idx_row, 0)

        def fire_gathers(b):
            for j in range(_BT_PER_W):
                dst = pl.ds(j * _GBLK, _GBLK)
                pltpu.async_copy(w_hbm.at[i0[b].at[j]], r0[b].at[dst], gsem[b])
                pltpu.async_copy(w_hbm.at[i1[b].at[j]], r1[b].at[dst], gsem[b])
                pltpu.async_copy(p_hbm.at[ip0[b].at[j]], p0b[b].at[dst], gsem[b])
                pltpu.async_copy(p_hbm.at[ip1[b].at[j]], p1b[b].at[dst], gsem[b])

        def wait_gathers(b):
            pltpu.make_async_copy(w_hbm.at[pl.ds(0, _CHUNK)], r0[b], gsem[b]).wait()
            pltpu.make_async_copy(w_hbm.at[pl.ds(0, _CHUNK)], r1[b], gsem[b]).wait()
            pltpu.make_async_copy(p_hbm.at[pl.ds(0, _CHUNK)], p0b[b], gsem[b]).wait()
            pltpu.make_async_copy(p_hbm.at[pl.ds(0, _CHUNK)], p1b[b], gsem[b]).wait()

        def fire_idx_load(l, b):
            pltpu.async_copy(
                idx_hbm.at[pl.ds(l * B + base_b, _CHUNK)], widx[b], isem[b])

        def wait_idx_load(b):
            pltpu.make_async_copy(
                idx_hbm.at[pl.ds(0, _CHUNK)], widx[b], isem[b]).wait()

        def mul(b):
            for k in range(_BT_PER_W):
                def mul_body(sub, carry3, k=k):
                    t0 = k * _GBLK + sub * _LANES
                    p0v = p0b[b][pl.ds(t0, _LANES)]
                    p1v = p1b[b][pl.ds(t0, _LANES)]
                    for u in range(_LANES):
                        t = t0 + u
                        vrow = r0[b][t] * p0v[u] + r1[b][t] * p1v[u]
                        pos = cvec + (k * (8 * _GBLK) + sub * _LANES + u)
                        plsc.store_scatter(obf[b], [pos], vrow)
                    return carry3

                lax.fori_loop(0, _GBLK // _LANES, mul_body, 0)

        def fire_writeout(l, b):
            for e2 in range(ne2):
                pltpu.async_copy(
                    obf[b].at[pl.ds(e2 * _HBLK, _HBLK)],
                    out_hbm.at[l * ne2 + e2, pl.ds(wid * _HBLK, _HBLK)],
                    wsem[b])

        def wait_writeout(b):
            pltpu.make_async_copy(
                obf[b], out_hbm.at[0, pl.ds(0, ne2 * _HBLK)], wsem[b]).wait()

        pltpu.sync_copy(idx_hbm.at[pl.ds(base_b, _CHUNK)], widx[0])
        idx_compute(0)
        fire_gathers(0)
        fire_idx_load(1, 1)

        def pair_body(i, carry):
            for b_off in range(nbuf):
                l = 2 * i + 1 + b_off
                sb = 1 - b_off
                mb = b_off

                @pl.when(l <= L - 1)
                def _stage():
                    wait_idx_load(sb)
                    idx_compute(sb)
                    fire_gathers(sb)

                @pl.when(l <= L - 2)
                def _prefetch():
                    fire_idx_load(l + 1, mb)

                wait_gathers(mb)

                @pl.when(l >= 3)
                def _drain():
                    wait_writeout(mb)

                mul(mb)
                fire_writeout(l - 1, mb)
            return carry

        lax.fori_loop(0, (L + 1) // 2, pair_body, 0)

        wait_writeout(0)
        wait_writeout(1)

    out = run(idx_t, W_full, p_flat)
    return (out.reshape(L, ne2, nbt, 8, _GBLK)
            .transpose(2, 4, 0, 1, 3)
            .reshape(B, L, _EMB))


# speedup vs baseline: 18.6005x; 1.2415x over previous
import functools

import jax
import jax.numpy as jnp
from jax import lax
from jax.experimental import pallas as pl
from jax.experimental.pallas import tpu as pltpu
from jax.experimental.pallas import tpu_sc as plsc

_WORD_COUNT = 1000000
_NUM_BUCKETS = 100000
_EMB = 16
_PRIMES = (65521, 60013)
_LANES = 16
_NW = 32
_GBLK = 128
_BT_PER_W = 4
_CHUNK = _BT_PER_W * _GBLK
_HBLK = 8 * _GBLK * _BT_PER_W


def _exact_mod(wf, prime):
    q = (wf * (1.0 / prime)).astype(jnp.int32).astype(jnp.float32)
    r = wf - q * float(prime)
    r = jnp.where(r < 0.0, r + float(prime), r)
    r = jnp.where(r >= float(prime), r - float(prime), r)
    return r.astype(jnp.int32)


def kernel(indices, W_full, p):
    B, L = indices.shape
    n = B * L
    nbt = B // _GBLK
    ne2 = _EMB // 8
    idx_t = indices.T.reshape(n)
    p_flat = p.T.reshape(-1)

    mesh = plsc.VectorSubcoreMesh(core_axis_name="c", subcore_axis_name="s")

    nbuf = 2
    scratch = (
        [pltpu.VMEM((_CHUNK,), jnp.int32) for _ in range(nbuf)]
        + [pltpu.VMEM((_CHUNK,), jnp.int32) for _ in range(4 * nbuf)]
        + [pltpu.VMEM((_CHUNK, _EMB), jnp.float32) for _ in range(2 * nbuf)]
        + [pltpu.VMEM((_CHUNK,), jnp.float32) for _ in range(2 * nbuf)]
        + [pltpu.VMEM((ne2 * _HBLK,), jnp.float32) for _ in range(nbuf)]
        + [pltpu.SemaphoreType.DMA for _ in range(3 * nbuf)]
    )

    @functools.partial(
        pl.kernel,
        mesh=mesh,
        compiler_params=pltpu.CompilerParams(
            use_tc_tiling_on_sc=False, needs_layout_passes=False),
        out_type=jax.ShapeDtypeStruct((L * ne2, nbt * 8 * _GBLK), jnp.float32),
        scratch_types=scratch,
    )
    def run(idx_hbm, w_hbm, p_hbm, out_hbm, *s):
        widx = s[0:2]
        i0 = s[2:4]
        i1 = s[4:6]
        ip0 = s[6:8]
        ip1 = s[8:10]
        r0 = s[10:12]
        r1 = s[12:14]
        p0b = s[14:16]
        p1b = s[16:18]
        obf = s[18:20]
        isem = s[20:22]
        gsem = s[22:24]
        wsem = s[24:26]

        wid = lax.axis_index("s") * 2 + lax.axis_index("c")
        base_b = wid * _CHUNK
        lane16 = lax.iota(jnp.int32, _LANES)
        cvec = (lane16 >> 3) * _HBLK + (lane16 & 7) * _GBLK

        def idx_compute(b):
            @plsc.parallel_loop(0, _CHUNK // _LANES, unroll=2)
            def idx_group(g):
                sl = pl.ds(g * _LANES, _LANES)
                w = widx[b][sl]
                wf = w.astype(jnp.float32)
                nz = w != 0
                a0 = jnp.where(nz, _exact_mod(wf, _PRIMES[0]), 0)
                a1 = jnp.where(nz, _exact_mod(wf, _PRIMES[1]), 0)
                wp = w + 3
                ap = jnp.where(wp >= _WORD_COUNT, wp - _WORD_COUNT, wp)
                i0[b][sl] = a0
                i1[b][sl] = a1
                ip0[b][sl] = ap
                ip1[b][sl] = ap + _WORD_COUNT

        def fire_gathers(b):
            for j in range(_BT_PER_W):
                sl = pl.ds(j * _GBLK, _GBLK)
                pltpu.async_copy(w_hbm.at[i0[b].at[sl]], r0[b].at[sl], gsem[b])
                pltpu.async_copy(w_hbm.at[i1[b].at[sl]], r1[b].at[sl], gsem[b])
                pltpu.async_copy(p_hbm.at[ip0[b].at[sl]], p0b[b].at[sl], gsem[b])
                pltpu.async_copy(p_hbm.at[ip1[b].at[sl]], p1b[b].at[sl], gsem[b])

        def wait_gathers(b):
            pltpu.make_async_copy(w_hbm.at[pl.ds(0, _CHUNK)], r0[b], gsem[b]).wait()
            pltpu.make_async_copy(w_hbm.at[pl.ds(0, _CHUNK)], r1[b], gsem[b]).wait()
            pltpu.make_async_copy(p_hbm.at[pl.ds(0, _CHUNK)], p0b[b], gsem[b]).wait()
            pltpu.make_async_copy(p_hbm.at[pl.ds(0, _CHUNK)], p1b[b], gsem[b]).wait()

        def fire_idx_load(l, b):
            pltpu.async_copy(
                idx_hbm.at[pl.ds(l * B + base_b, _CHUNK)], widx[b], isem[b])

        def wait_idx_load(b):
            pltpu.make_async_copy(
                idx_hbm.at[pl.ds(0, _CHUNK)], widx[b], isem[b]).wait()

        def mul(b):
            @plsc.parallel_loop(0, _CHUNK // _LANES, unroll=2)
            def mul_sub(sub):
                t0 = sub * _LANES
                p0v = p0b[b][pl.ds(t0, _LANES)]
                p1v = p1b[b][pl.ds(t0, _LANES)]
                base = (sub >> 3) * (8 * _GBLK) + (sub & 7) * _LANES
                for u in range(_LANES):
                    vrow = r0[b][t0 + u] * p0v[u] + r1[b][t0 + u] * p1v[u]
                    plsc.store_scatter(obf[b], [cvec + (base + u)], vrow)

        def fire_writeout(l, b):
            for e2 in range(ne2):
                pltpu.async_copy(
                    obf[b].at[pl.ds(e2 * _HBLK, _HBLK)],
                    out_hbm.at[l * ne2 + e2, pl.ds(wid * _HBLK, _HBLK)],
                    wsem[b])

        def wait_writeout(b):
            pltpu.make_async_copy(
                obf[b], out_hbm.at[0, pl.ds(0, ne2 * _HBLK)], wsem[b]).wait()

        pltpu.sync_copy(idx_hbm.at[pl.ds(base_b, _CHUNK)], widx[0])
        idx_compute(0)
        fire_gathers(0)
        fire_idx_load(1, 1)

        def pair_body(i, carry):
            for b_off in range(nbuf):
                l = 2 * i + 1 + b_off
                sb = 1 - b_off
                mb = b_off

                @pl.when(l <= L - 1)
                def _stage():
                    wait_idx_load(sb)
                    idx_compute(sb)
                    fire_gathers(sb)

                @pl.when(l <= L - 2)
                def _prefetch():
                    fire_idx_load(l + 1, mb)

                wait_gathers(mb)

                @pl.when(l >= 3)
                def _drain():
                    wait_writeout(mb)

                mul(mb)
                fire_writeout(l - 1, mb)
            return carry

        lax.fori_loop(0, (L + 1) // 2, pair_body, 0)

        wait_writeout(0)
        wait_writeout(1)

    out = run(idx_t, W_full, p_flat)
    return (out.reshape(L, ne2, nbt, 8, _GBLK)
            .transpose(2, 4, 0, 1, 3)
            .reshape(B, L, _EMB))


# speedup vs baseline: 19.6461x; 1.0562x over previous
import functools

import jax
import jax.numpy as jnp
from jax import lax
from jax.experimental import pallas as pl
from jax.experimental.pallas import tpu as pltpu
from jax.experimental.pallas import tpu_sc as plsc

_WORD_COUNT = 1000000
_NUM_BUCKETS = 100000
_EMB = 16
_PRIMES = (65521, 60013)
_LANES = 16
_NW = 32
_GBLK = 128
_BT_PER_W = 4
_CHUNK = _BT_PER_W * _GBLK
_HBLK = 8 * _GBLK * _BT_PER_W


def _exact_mod(wf, prime):
    q = (wf * (1.0 / prime)).astype(jnp.int32).astype(jnp.float32)
    r = wf - q * float(prime)
    r = jnp.where(r < 0.0, r + float(prime), r)
    r = jnp.where(r >= float(prime), r - float(prime), r)
    return r.astype(jnp.int32)


def kernel(indices, W_full, p):
    B, L = indices.shape
    n = B * L
    nbt = B // _GBLK
    ne2 = _EMB // 8
    idx_t = indices.T.reshape(n)
    p_flat = p.T.reshape(-1)

    mesh = plsc.VectorSubcoreMesh(core_axis_name="c", subcore_axis_name="s")

    nbuf = 2
    scratch = (
        [pltpu.VMEM((_CHUNK,), jnp.int32) for _ in range(nbuf)]
        + [pltpu.VMEM((_CHUNK,), jnp.int32) for _ in range(4 * nbuf)]
        + [pltpu.VMEM((_CHUNK, _EMB), jnp.float32) for _ in range(2 * nbuf)]
        + [pltpu.VMEM((_CHUNK,), jnp.float32) for _ in range(2 * nbuf)]
        + [pltpu.VMEM((ne2 * _HBLK,), jnp.float32) for _ in range(nbuf)]
        + [pltpu.SemaphoreType.DMA for _ in range(3 * nbuf)]
    )

    @functools.partial(
        pl.kernel,
        mesh=mesh,
        compiler_params=pltpu.CompilerParams(
            use_tc_tiling_on_sc=False, needs_layout_passes=False),
        out_type=jax.ShapeDtypeStruct((L * ne2, nbt * 8 * _GBLK), jnp.float32),
        scratch_types=scratch,
    )
    def run(idx_hbm, w_hbm, p_hbm, out_hbm, *s):
        widx = s[0:2]
        i0 = s[2:4]
        i1 = s[4:6]
        ip0 = s[6:8]
        ip1 = s[8:10]
        r0 = s[10:12]
        r1 = s[12:14]
        p0b = s[14:16]
        p1b = s[16:18]
        obf = s[18:20]
        isem = s[20:22]
        gsem = s[22:24]
        wsem = s[24:26]

        wid = lax.axis_index("s") * 2 + lax.axis_index("c")
        base_b = wid * _CHUNK
        lane16 = lax.iota(jnp.int32, _LANES)
        cvec = (lane16 >> 3) * _HBLK + (lane16 & 7) * _GBLK

        def idx_compute(b):
            @plsc.parallel_loop(0, _CHUNK // _LANES, unroll=4)
            def idx_group(g):
                sl = pl.ds(g * _LANES, _LANES)
                w = widx[b][sl]
                wf = w.astype(jnp.float32)
                nz = w != 0
                a0 = jnp.where(nz, _exact_mod(wf, _PRIMES[0]), 0)
                a1 = jnp.where(nz, _exact_mod(wf, _PRIMES[1]), 0)
                wp = w + 3
                ap = jnp.where(wp >= _WORD_COUNT, wp - _WORD_COUNT, wp)
                i0[b][sl] = a0
                i1[b][sl] = a1
                ip0[b][sl] = ap
                ip1[b][sl] = ap + _WORD_COUNT

        def fire_gathers(b):
            for j in range(_BT_PER_W):
                sl = pl.ds(j * _GBLK, _GBLK)
                pltpu.async_copy(w_hbm.at[i0[b].at[sl]], r0[b].at[sl], gsem[b])
                pltpu.async_copy(w_hbm.at[i1[b].at[sl]], r1[b].at[sl], gsem[b])
                pltpu.async_copy(p_hbm.at[ip0[b].at[sl]], p0b[b].at[sl], gsem[b])
                pltpu.async_copy(p_hbm.at[ip1[b].at[sl]], p1b[b].at[sl], gsem[b])

        def wait_gathers(b):
            pltpu.make_async_copy(w_hbm.at[pl.ds(0, _CHUNK)], r0[b], gsem[b]).wait()
            pltpu.make_async_copy(w_hbm.at[pl.ds(0, _CHUNK)], r1[b], gsem[b]).wait()
            pltpu.make_async_copy(p_hbm.at[pl.ds(0, _CHUNK)], p0b[b], gsem[b]).wait()
            pltpu.make_async_copy(p_hbm.at[pl.ds(0, _CHUNK)], p1b[b], gsem[b]).wait()

        def fire_idx_load(l, b):
            pltpu.async_copy(
                idx_hbm.at[pl.ds(l * B + base_b, _CHUNK)], widx[b], isem[b])

        def wait_idx_load(b):
            pltpu.make_async_copy(
                idx_hbm.at[pl.ds(0, _CHUNK)], widx[b], isem[b]).wait()

        def mul(b):
            @plsc.parallel_loop(0, _CHUNK // _LANES, unroll=4)
            def mul_sub(sub):
                t0 = sub * _LANES
                p0v = p0b[b][pl.ds(t0, _LANES)]
                p1v = p1b[b][pl.ds(t0, _LANES)]
                base = (sub >> 3) * (8 * _GBLK) + (sub & 7) * _LANES
                for u in range(_LANES):
                    vrow = r0[b][t0 + u] * p0v[u] + r1[b][t0 + u] * p1v[u]
                    plsc.store_scatter(obf[b], [cvec + (base + u)], vrow)

        def fire_writeout(l, b):
            for e2 in range(ne2):
                pltpu.async_copy(
                    obf[b].at[pl.ds(e2 * _HBLK, _HBLK)],
                    out_hbm.at[l * ne2 + e2, pl.ds(wid * _HBLK, _HBLK)],
                    wsem[b])

        def wait_writeout(b):
            pltpu.make_async_copy(
                obf[b], out_hbm.at[0, pl.ds(0, ne2 * _HBLK)], wsem[b]).wait()

        pltpu.sync_copy(idx_hbm.at[pl.ds(base_b, _CHUNK)], widx[0])
        idx_compute(0)
        fire_gathers(0)
        fire_idx_load(1, 1)

        def pair_body(i, carry):
            for b_off in range(nbuf):
                l = 2 * i + 1 + b_off
                sb = 1 - b_off
                mb = b_off

                @pl.when(l <= L - 1)
                def _stage():
                    wait_idx_load(sb)
                    idx_compute(sb)
                    fire_gathers(sb)

                @pl.when(l <= L - 2)
                def _prefetch():
                    fire_idx_load(l + 1, mb)

                wait_gathers(mb)

                @pl.when(l >= 3)
                def _drain():
                    wait_writeout(mb)

                mul(mb)
                fire_writeout(l - 1, mb)
            return carry

        lax.fori_loop(0, (L + 1) // 2, pair_body, 0)

        wait_writeout(0)
        wait_writeout(1)

    out = run(idx_t, W_full, p_flat)
    return (out.reshape(L, ne2, nbt, 8, _GBLK)
            .transpose(2, 4, 0, 1, 3)
            .reshape(B, L, _EMB))


# speedup vs baseline: 21.4884x; 1.0938x over previous
import functools

import jax
import jax.numpy as jnp
from jax import lax
from jax.experimental import pallas as pl
from jax.experimental.pallas import tpu as pltpu
from jax.experimental.pallas import tpu_sc as plsc

_WORD_COUNT = 1000000
_NUM_BUCKETS = 100000
_EMB = 16
_PRIMES = (65521, 60013)
_LANES = 16
_NW = 32
_GBLK = 128
_BT_PER_W = 4
_CHUNK = _BT_PER_W * _GBLK
_HBLK = 8 * _GBLK * _BT_PER_W


def _exact_mod(wf, prime):
    q = (wf * (1.0 / prime)).astype(jnp.int32).astype(jnp.float32)
    r = wf - q * float(prime)
    r = jnp.where(r < 0.0, r + float(prime), r)
    r = jnp.where(r >= float(prime), r - float(prime), r)
    return r.astype(jnp.int32)


def kernel(indices, W_full, p):
    B, L = indices.shape
    n = B * L
    nbt = B // _GBLK
    ne2 = _EMB // 8
    idx_t = indices.T.reshape(n)
    p_flat = p.T.reshape(-1)

    mesh = plsc.VectorSubcoreMesh(core_axis_name="c", subcore_axis_name="s")

    nbuf = 2
    scratch = (
        [pltpu.VMEM((_CHUNK,), jnp.int32) for _ in range(nbuf)]
        + [pltpu.VMEM((_CHUNK,), jnp.int32) for _ in range(4 * nbuf)]
        + [pltpu.VMEM((_CHUNK, _EMB), jnp.float32) for _ in range(2 * nbuf)]
        + [pltpu.VMEM((_CHUNK,), jnp.float32) for _ in range(2 * nbuf)]
        + [pltpu.VMEM((ne2 * _HBLK,), jnp.float32) for _ in range(nbuf)]
        + [pltpu.SemaphoreType.DMA for _ in range(3 * nbuf)]
    )

    @functools.partial(
        pl.kernel,
        mesh=mesh,
        compiler_params=pltpu.CompilerParams(
            use_tc_tiling_on_sc=False, needs_layout_passes=False),
        out_type=jax.ShapeDtypeStruct((L * ne2, nbt * 8 * _GBLK), jnp.float32),
        scratch_types=scratch,
    )
    def run(idx_hbm, w_hbm, p_hbm, out_hbm, *s):
        widx = s[0:2]
        i0 = s[2:4]
        i1 = s[4:6]
        ip0 = s[6:8]
        ip1 = s[8:10]
        r0 = s[10:12]
        r1 = s[12:14]
        p0b = s[14:16]
        p1b = s[16:18]
        obf = s[18:20]
        isem = s[20:22]
        gsem = s[22:24]
        wsem = s[24:26]

        wid = lax.axis_index("s") * 2 + lax.axis_index("c")
        base_b = wid * _CHUNK
        lane16 = lax.iota(jnp.int32, _LANES)
        cvec = (lane16 >> 3) * _HBLK + (lane16 & 7) * _GBLK

        def idx_compute(b):
            @plsc.parallel_loop(0, _CHUNK // _LANES, unroll=8)
            def idx_group(g):
                sl = pl.ds(g * _LANES, _LANES)
                w = widx[b][sl]
                wf = w.astype(jnp.float32)
                nz = w != 0
                a0 = jnp.where(nz, _exact_mod(wf, _PRIMES[0]), 0)
                a1 = jnp.where(nz, _exact_mod(wf, _PRIMES[1]), 0)
                wp = w + 3
                ap = jnp.where(wp >= _WORD_COUNT, wp - _WORD_COUNT, wp)
                i0[b][sl] = a0
                i1[b][sl] = a1
                ip0[b][sl] = ap
                ip1[b][sl] = ap + _WORD_COUNT

        def fire_gathers(b):
            for j in range(_BT_PER_W):
                sl = pl.ds(j * _GBLK, _GBLK)
                pltpu.async_copy(w_hbm.at[i0[b].at[sl]], r0[b].at[sl], gsem[b])
                pltpu.async_copy(w_hbm.at[i1[b].at[sl]], r1[b].at[sl], gsem[b])
                pltpu.async_copy(p_hbm.at[ip0[b].at[sl]], p0b[b].at[sl], gsem[b])
                pltpu.async_copy(p_hbm.at[ip1[b].at[sl]], p1b[b].at[sl], gsem[b])

        def wait_gathers(b):
            pltpu.make_async_copy(w_hbm.at[pl.ds(0, _CHUNK)], r0[b], gsem[b]).wait()
            pltpu.make_async_copy(w_hbm.at[pl.ds(0, _CHUNK)], r1[b], gsem[b]).wait()
            pltpu.make_async_copy(p_hbm.at[pl.ds(0, _CHUNK)], p0b[b], gsem[b]).wait()
            pltpu.make_async_copy(p_hbm.at[pl.ds(0, _CHUNK)], p1b[b], gsem[b]).wait()

        def fire_idx_load(l, b):
            pltpu.async_copy(
                idx_hbm.at[pl.ds(l * B + base_b, _CHUNK)], widx[b], isem[b])

        def wait_idx_load(b):
            pltpu.make_async_copy(
                idx_hbm.at[pl.ds(0, _CHUNK)], widx[b], isem[b]).wait()

        def mul(b):
            @plsc.parallel_loop(0, _CHUNK // _LANES, unroll=8)
            def mul_sub(sub):
                t0 = sub * _LANES
                p0v = p0b[b][pl.ds(t0, _LANES)]
                p1v = p1b[b][pl.ds(t0, _LANES)]
                base = (sub >> 3) * (8 * _GBLK) + (sub & 7) * _LANES
                for u in range(_LANES):
                    vrow = r0[b][t0 + u] * p0v[u] + r1[b][t0 + u] * p1v[u]
                    plsc.store_scatter(obf[b], [cvec + (base + u)], vrow)

        def fire_writeout(l, b):
            for e2 in range(ne2):
                pltpu.async_copy(
                    obf[b].at[pl.ds(e2 * _HBLK, _HBLK)],
                    out_hbm.at[l * ne2 + e2, pl.ds(wid * _HBLK, _HBLK)],
                    wsem[b])

        def wait_writeout(b):
            pltpu.make_async_copy(
                obf[b], out_hbm.at[0, pl.ds(0, ne2 * _HBLK)], wsem[b]).wait()

        pltpu.sync_copy(idx_hbm.at[pl.ds(base_b, _CHUNK)], widx[0])
        idx_compute(0)
        fire_gathers(0)
        fire_idx_load(1, 1)

        def pair_body(i, carry):
            for b_off in range(nbuf):
                l = 2 * i + 1 + b_off
                sb = 1 - b_off
                mb = b_off

                @pl.when(l <= L - 1)
                def _stage():
                    wait_idx_load(sb)
                    idx_compute(sb)
                    fire_gathers(sb)

                @pl.when(l <= L - 2)
                def _prefetch():
                    fire_idx_load(l + 1, mb)

                wait_gathers(mb)

                @pl.when(l >= 3)
                def _drain():
                    wait_writeout(mb)

                mul(mb)
                fire_writeout(l - 1, mb)
            return carry

        lax.fori_loop(0, (L + 1) // 2, pair_body, 0)

        wait_writeout(0)
        wait_writeout(1)

    out = run(idx_t, W_full, p_flat)
    return (out.reshape(L, ne2, nbt, 8, _GBLK)
            .transpose(2, 4, 0, 1, 3)
            .reshape(B, L, _EMB))
